# baseline (speedup 1.0000x reference)
import math

import jax
import jax.numpy as jnp
from jax.experimental import pallas as pl
from jax.experimental.pallas import tpu as pltpu


def _pe_add_kernel(scale, x_ref, pe_ref, o_ref):
    o_ref[...] = x_ref[...] * scale + pe_ref[...]


def kernel(x, pe):
    B, T, H, W, S = x.shape
    HW = H * W
    scale = math.sqrt(S)

    x4 = x.reshape(B, T, HW, S)
    pe3 = pe.reshape(pe.shape[0], 1, S)

    out = pl.pallas_call(
        lambda x_ref, pe_ref, o_ref: _pe_add_kernel(scale, x_ref, pe_ref, o_ref),
        grid=(B, T),
        in_specs=[
            pl.BlockSpec((1, 1, HW, S), lambda b, t: (b, t, 0, 0)),
            pl.BlockSpec((1, 1, S), lambda b, t: (t, 0, 0)),
        ],
        out_specs=pl.BlockSpec((1, 1, HW, S), lambda b, t: (b, t, 0, 0)),
        out_shape=jax.ShapeDtypeStruct((B, T, HW, S), x.dtype),
        compiler_params=pltpu.CompilerParams(
            dimension_semantics=("parallel", "parallel"),
        ),
    )(x4, pe3)

    return out.reshape(B, T, H, W, S)


# speedup vs baseline: 1.5308x; 1.5308x over previous
import math

import jax
import jax.numpy as jnp
from jax.experimental import pallas as pl
from jax.experimental.pallas import tpu as pltpu

_BLK = 8


def _pe_add_kernel(scale, x_ref, pe_ref, o_ref):
    o_ref[...] = x_ref[...] * scale + pe_ref[...]


def kernel(x, pe):
    B, T, H, W, S = x.shape
    HW = H * W
    R = B * T
    scale = math.sqrt(S)

    x3 = x.reshape(R, HW, S)
    pe3 = jnp.tile(pe[:T], (B, 1)).reshape(R, 1, S)

    blk = _BLK if R % _BLK == 0 else 1
    out = pl.pallas_call(
        lambda x_ref, pe_ref, o_ref: _pe_add_kernel(scale, x_ref, pe_ref, o_ref),
        grid=(R // blk,),
        in_specs=[
            pl.BlockSpec((blk, HW, S), lambda i: (i, 0, 0)),
            pl.BlockSpec((blk, 1, S), lambda i: (i, 0, 0)),
        ],
        out_specs=pl.BlockSpec((blk, HW, S), lambda i: (i, 0, 0)),
        out_shape=jax.ShapeDtypeStruct((R, HW, S), x.dtype),
        compiler_params=pltpu.CompilerParams(
            dimension_semantics=("parallel",),
        ),
    )(x3, pe3)

    return out.reshape(B, T, H, W, S)


# speedup vs baseline: 1.5401x; 1.0060x over previous
import math

import jax
import jax.numpy as jnp
from jax.experimental import pallas as pl
from jax.experimental.pallas import tpu as pltpu


def _pe_add_kernel(scale, x_ref, pe_ref, o_ref):
    o_ref[...] = x_ref[...] * scale + pe_ref[...]


def kernel(x, pe):
    B, T, H, W, S = x.shape
    HW = H * W
    scale = math.sqrt(S)

    x4 = x.reshape(B, T, HW, S)
    pe3 = pe[:T].reshape(T, 1, S)

    out = pl.pallas_call(
        lambda x_ref, pe_ref, o_ref: _pe_add_kernel(scale, x_ref, pe_ref, o_ref),
        grid=(B,),
        in_specs=[
            pl.BlockSpec((1, T, HW, S), lambda b: (b, 0, 0, 0)),
            pl.BlockSpec((T, 1, S), lambda b: (0, 0, 0)),
        ],
        out_specs=pl.BlockSpec((1, T, HW, S), lambda b: (b, 0, 0, 0)),
        out_shape=jax.ShapeDtypeStruct((B, T, HW, S), x.dtype),
        compiler_params=pltpu.CompilerParams(
            dimension_semantics=("parallel",),
        ),
    )(x4, pe3)

    return out.reshape(B, T, H, W, S)
